# baseline (speedup 1.0000x reference)
import functools

import jax
import jax.numpy as jnp
from jax import lax
from jax.experimental import pallas as pl
from jax.experimental.pallas import tpu as pltpu
from jax.experimental.pallas import tpu_sc as plsc

EMBED_DIM = 32
NUM_CORES = 2
NUM_SUBCORES = 16
NW = NUM_CORES * NUM_SUBCORES


def _make_gather(nb, h):
    assert nb % (128 * NW) == 0 and EMBED_DIM == 32
    cpw = nb // 128 // NW
    unit = 128 * cpw
    n = h
    assert n >= 6

    mesh = plsc.VectorSubcoreMesh(
        core_axis_name="c", subcore_axis_name="s",
        num_cores=NUM_CORES, num_subcores=NUM_SUBCORES)

    scratch = (
        [pltpu.VMEM((unit,), jnp.int32) for _ in range(2)]
        + [pltpu.VMEM((unit, EMBED_DIM), jnp.float32) for _ in range(2)]
        + [pltpu.VMEM((4, cpw, 8, 128), jnp.float32) for _ in range(2)]
        + [pltpu.SemaphoreType.DMA for _ in range(6)]
    )

    @functools.partial(
        pl.kernel,
        out_type=jax.ShapeDtypeStruct((h, 4, nb // 128, 8, 128),
                                      jnp.float32),
        mesh=mesh,
        scratch_types=scratch,
        compiler_params=pltpu.CompilerParams(
            use_tc_tiling_on_sc=False, needs_layout_passes=False),
    )
    def gather_kernel(idx_hbm, table_hbm, out_hbm, *refs):
        idx_v = refs[0:2]
        rows_v = refs[2:4]
        t_v = refs[4:6]
        idx_sem = refs[6:8]
        gat_sem = refs[8:10]
        out_sem = refs[10:12]

        wid = lax.axis_index("s") * NUM_CORES + lax.axis_index("c")
        c0 = wid * cpw
        b0 = wid * unit
        iota = lax.iota(jnp.int32, 16)

        def idx_copy(i, b):
            pltpu.async_copy(
                idx_hbm.at[pl.ds(i * nb + b0, unit)], idx_v[b], idx_sem[b])

        def wait_idx(b):
            pltpu.make_async_copy(
                idx_hbm.at[pl.ds(0, unit)], idx_v[b], idx_sem[b]).wait()

        def gather_start(b):
            pltpu.async_copy(table_hbm.at[idx_v[b]], rows_v[b], gat_sem[b])

        def wait_gat(b):
            pltpu.make_async_copy(
                table_hbm.at[idx_v[b]], rows_v[b], gat_sem[b]).wait()

        def out_copy(i, b):
            for t in range(4):
                pltpu.async_copy(
                    t_v[b].at[t], out_hbm.at[i, t, pl.ds(c0, cpw)],
                    out_sem[b])

        def wait_out(b):
            for t in range(4):
                pltpu.make_async_copy(
                    t_v[b].at[t], out_hbm.at[0, t, pl.ds(0, cpw)],
                    out_sem[b]).wait()

        def transpose(b):
            def body(cb, carry):
                rids = [cb * 128 + l0 * 16 + iota for l0 in range(8)]
                for t in range(4):
                    for s in range(8):
                        col = jnp.zeros((16,), jnp.int32) + (8 * t + s)
                        for l0 in range(8):
                            v = plsc.load_gather(rows_v[b], [rids[l0], col])
                            t_v[b][t, cb, s, pl.ds(l0 * 16, 16)] = v
                return carry
            lax.fori_loop(0, cpw, body, 0)

        def step_a(i, b):
            wait_idx(b)
            gather_start(b)

        def step_b(j, pb, check_out, prefetch):
            wait_gat(pb)
            if check_out:
                wait_out(pb)
            transpose(pb)
            out_copy(j, pb)
            if prefetch:
                idx_copy(j + 2, pb)

        idx_copy(0, 0)
        idx_copy(1, 1)
        step_a(0, 0)
        step_a(1, 1)
        step_b(0, 0, False, True)
        step_a(2, 0)
        step_b(1, 1, False, True)

        def group(g, carry):
            i0 = 3 + 2 * g
            step_a(i0, 1)
            step_b(i0 - 1, 0, True, True)
            step_a(i0 + 1, 0)
            step_b(i0, 1, True, True)
            return carry

        lax.fori_loop(0, (n - 4) // 2, group, 0)

        step_a(n - 1, 1)
        step_b(n - 2, 0, True, False)
        step_b(n - 1, 1, True, False)
        wait_out(0)
        wait_out(1)

    return gather_kernel


def kernel(batch, word_embeddings):
    nb, h = batch.shape
    flat = batch.T.reshape(nb * h)
    L = _make_gather(nb, h)(flat, word_embeddings)
    return jnp.transpose(L, (2, 4, 0, 1, 3)).reshape(nb, h, EMBED_DIM)


# speedup vs baseline: 2.5681x; 2.5681x over previous
import functools

import jax
import jax.numpy as jnp
from jax import lax
from jax.experimental import pallas as pl
from jax.experimental.pallas import tpu as pltpu
from jax.experimental.pallas import tpu_sc as plsc

EMBED_DIM = 32
NUM_CORES = 2
NUM_SUBCORES = 16
NW = NUM_CORES * NUM_SUBCORES
SPAD = 10
LPAD = 129


def _make_gather(nb, h):
    assert nb % (128 * NW) == 0 and EMBED_DIM == 32
    cpw = nb // 128 // NW
    unit = 128 * cpw
    n = h
    assert n >= 6

    mesh = plsc.VectorSubcoreMesh(
        core_axis_name="c", subcore_axis_name="s",
        num_cores=NUM_CORES, num_subcores=NUM_SUBCORES)

    scratch = (
        [pltpu.VMEM((unit,), jnp.int32) for _ in range(2)]
        + [pltpu.VMEM((unit, EMBED_DIM), jnp.float32) for _ in range(2)]
        + [pltpu.VMEM((4, cpw, SPAD, LPAD), jnp.float32) for _ in range(2)]
        + [pltpu.SemaphoreType.DMA for _ in range(6)]
    )

    @functools.partial(
        pl.kernel,
        out_type=jax.ShapeDtypeStruct((h, 4, nb // 128, 8, 128),
                                      jnp.float32),
        mesh=mesh,
        scratch_types=scratch,
        compiler_params=pltpu.CompilerParams(
            use_tc_tiling_on_sc=False, needs_layout_passes=False),
    )
    def gather_kernel(idx_hbm, table_hbm, out_hbm, *refs):
        idx_v = refs[0:2]
        rows_v = refs[2:4]
        t_v = refs[4:6]
        idx_sem = refs[6:8]
        gat_sem = refs[8:10]
        out_sem = refs[10:12]

        wid = lax.axis_index("s") * NUM_CORES + lax.axis_index("c")
        c0 = wid * cpw
        b0 = wid * unit

        iota = lax.iota(jnp.int32, 16)
        tv_lo = lax.shift_right_logical(iota, 3)
        tv_hi = tv_lo + 2
        sv = lax.bitwise_and(iota, 7)

        def idx_copy(i, b):
            pltpu.async_copy(
                idx_hbm.at[pl.ds(i * nb + b0, unit)], idx_v[b], idx_sem[b])

        def wait_idx(b):
            pltpu.make_async_copy(
                idx_hbm.at[pl.ds(0, unit)], idx_v[b], idx_sem[b]).wait()

        def gather_start(b):
            pltpu.async_copy(table_hbm.at[idx_v[b]], rows_v[b], gat_sem[b])

        def wait_gat(b):
            pltpu.make_async_copy(
                table_hbm.at[idx_v[b]], rows_v[b], gat_sem[b]).wait()

        def out_copy(i, b):
            for t in range(4):
                for cb in range(cpw):
                    pltpu.async_copy(
                        t_v[b].at[t, cb, pl.ds(0, 8), pl.ds(0, 128)],
                        out_hbm.at[i, t, c0 + cb], out_sem[b])

        def wait_out(b):
            for t in range(4):
                for cb in range(cpw):
                    pltpu.make_async_copy(
                        t_v[b].at[t, cb, pl.ds(0, 8), pl.ds(0, 128)],
                        out_hbm.at[0, t, 0], out_sem[b]).wait()

        def transpose(b):
            def body(k, carry):
                cb = k // 16
                cbv = jnp.zeros((16,), jnp.int32) + cb
                for u in range(8):
                    rr = (k % 16) * 8 + u
                    r = cb * 128 + rr
                    lv = jnp.zeros((16,), jnp.int32) + rr
                    v_lo = rows_v[b][r, pl.ds(0, 16)]
                    v_hi = rows_v[b][r, pl.ds(16, 16)]
                    plsc.store_scatter(t_v[b], [tv_lo, cbv, sv, lv], v_lo)
                    plsc.store_scatter(t_v[b], [tv_hi, cbv, sv, lv], v_hi)
                return carry
            lax.fori_loop(0, cpw * 16, body, 0)

        def step_a(i, b):
            wait_idx(b)
            gather_start(b)

        def step_b(j, pb, check_out, prefetch):
            wait_gat(pb)
            if check_out:
                wait_out(pb)
            transpose(pb)
            out_copy(j, pb)
            if prefetch:
                idx_copy(j + 2, pb)

        idx_copy(0, 0)
        idx_copy(1, 1)
        step_a(0, 0)
        step_a(1, 1)
        step_b(0, 0, False, True)
        step_a(2, 0)
        step_b(1, 1, False, True)

        def group(g, carry):
            i0 = 3 + 2 * g
            step_a(i0, 1)
            step_b(i0 - 1, 0, True, True)
            step_a(i0 + 1, 0)
            step_b(i0, 1, True, True)
            return carry

        lax.fori_loop(0, (n - 4) // 2, group, 0)

        step_a(n - 1, 1)
        step_b(n - 2, 0, True, False)
        step_b(n - 1, 1, True, False)
        wait_out(0)
        wait_out(1)

    return gather_kernel


def kernel(batch, word_embeddings):
    nb, h = batch.shape
    flat = batch.T.reshape(nb * h)
    L = _make_gather(nb, h)(flat, word_embeddings)
    return jnp.transpose(L, (2, 4, 0, 1, 3)).reshape(nb, h, EMBED_DIM)


# speedup vs baseline: 2.5967x; 1.0111x over previous
import functools

import jax
import jax.numpy as jnp
from jax import lax
from jax.experimental import pallas as pl
from jax.experimental.pallas import tpu as pltpu
from jax.experimental.pallas import tpu_sc as plsc

EMBED_DIM = 32
NUM_CORES = 2
NUM_SUBCORES = 16
NW = NUM_CORES * NUM_SUBCORES
SPAD = 10
LPAD = 129


def _make_gather(nb, h):
    assert nb % (128 * NW) == 0 and EMBED_DIM == 32
    cpw = nb // 128 // NW
    unit = 128 * cpw
    n = h
    assert n >= 6

    mesh = plsc.VectorSubcoreMesh(
        core_axis_name="c", subcore_axis_name="s",
        num_cores=NUM_CORES, num_subcores=NUM_SUBCORES)

    scratch = (
        [pltpu.VMEM((unit,), jnp.int32) for _ in range(2)]
        + [pltpu.VMEM((unit, EMBED_DIM), jnp.float32) for _ in range(2)]
        + [pltpu.VMEM((4, cpw, SPAD, LPAD), jnp.float32) for _ in range(2)]
        + [pltpu.SemaphoreType.DMA for _ in range(6)]
    )

    @functools.partial(
        pl.kernel,
        out_type=jax.ShapeDtypeStruct((h, 4, nb // 128, 8, 128),
                                      jnp.float32),
        mesh=mesh,
        scratch_types=scratch,
        compiler_params=pltpu.CompilerParams(
            use_tc_tiling_on_sc=False, needs_layout_passes=False),
    )
    def gather_kernel(idx_hbm, table_hbm, out_hbm, *refs):
        idx_v = refs[0:2]
        rows_v = refs[2:4]
        t_v = refs[4:6]
        idx_sem = refs[6:8]
        gat_sem = refs[8:10]
        out_sem = refs[10:12]

        wid = lax.axis_index("s") * NUM_CORES + lax.axis_index("c")
        c0 = wid * cpw
        b0 = wid * unit

        iota = lax.iota(jnp.int32, 16)
        tv_lo = lax.shift_right_logical(iota, 3)
        tv_hi = tv_lo + 2
        sv = lax.bitwise_and(iota, 7)

        def idx_copy(i, b):
            pltpu.async_copy(
                idx_hbm.at[pl.ds(i * nb + b0, unit)], idx_v[b], idx_sem[b])

        def wait_idx(b):
            pltpu.make_async_copy(
                idx_hbm.at[pl.ds(0, unit)], idx_v[b], idx_sem[b]).wait()

        def gather_start(b):
            pltpu.async_copy(table_hbm.at[idx_v[b]], rows_v[b], gat_sem[b])

        def wait_gat(b):
            pltpu.make_async_copy(
                table_hbm.at[idx_v[b]], rows_v[b], gat_sem[b]).wait()

        def out_copy(i, b):
            for t in range(4):
                pltpu.async_copy(
                    t_v[b].at[t, pl.ds(0, cpw), pl.ds(0, 8), pl.ds(0, 128)],
                    out_hbm.at[i, t, pl.ds(c0, cpw)], out_sem[b])

        def wait_out(b):
            for t in range(4):
                pltpu.make_async_copy(
                    t_v[b].at[t, pl.ds(0, cpw), pl.ds(0, 8), pl.ds(0, 128)],
                    out_hbm.at[0, t, pl.ds(0, cpw)], out_sem[b]).wait()

        def transpose(b):
            def body(k, carry):
                cb = k // 8
                cbv = jnp.zeros((16,), jnp.int32) + cb
                for u in range(16):
                    rr = (k % 8) * 16 + u
                    r = cb * 128 + rr
                    lv = jnp.zeros((16,), jnp.int32) + rr
                    v_lo = rows_v[b][r, pl.ds(0, 16)]
                    v_hi = rows_v[b][r, pl.ds(16, 16)]
                    plsc.store_scatter(t_v[b], [tv_lo, cbv, sv, lv], v_lo)
                    plsc.store_scatter(t_v[b], [tv_hi, cbv, sv, lv], v_hi)
                return carry
            lax.fori_loop(0, cpw * 8, body, 0)

        def step_a(i, b):
            wait_idx(b)
            gather_start(b)

        def step_b(j, pb, check_out, prefetch):
            wait_gat(pb)
            if check_out:
                wait_out(pb)
            transpose(pb)
            out_copy(j, pb)
            if prefetch:
                idx_copy(j + 2, pb)

        idx_copy(0, 0)
        idx_copy(1, 1)
        step_a(0, 0)
        step_a(1, 1)
        step_b(0, 0, False, True)
        step_a(2, 0)
        step_b(1, 1, False, True)

        def group(g, carry):
            i0 = 3 + 2 * g
            step_a(i0, 1)
            step_b(i0 - 1, 0, True, True)
            step_a(i0 + 1, 0)
            step_b(i0, 1, True, True)
            return carry

        lax.fori_loop(0, (n - 4) // 2, group, 0)

        step_a(n - 1, 1)
        step_b(n - 2, 0, True, False)
        step_b(n - 1, 1, True, False)
        wait_out(0)
        wait_out(1)

    return gather_kernel


def kernel(batch, word_embeddings):
    nb, h = batch.shape
    flat = batch.T.reshape(nb * h)
    L = _make_gather(nb, h)(flat, word_embeddings)
    return jnp.transpose(L, (2, 4, 0, 1, 3)).reshape(nb, h, EMBED_DIM)


# speedup vs baseline: 2.6040x; 1.0028x over previous
import functools

import jax
import jax.numpy as jnp
from jax import lax
from jax.experimental import pallas as pl
from jax.experimental.pallas import tpu as pltpu
from jax.experimental.pallas import tpu_sc as plsc

EMBED_DIM = 32
NUM_CORES = 2
NUM_SUBCORES = 16
NW = NUM_CORES * NUM_SUBCORES
SPAD = 10
LPAD = 129


def _make_gather(nb, h):
    assert nb % (128 * NW) == 0 and EMBED_DIM == 32
    cpw = nb // 128 // NW
    unit = 128 * cpw
    n = h
    assert n >= 6

    mesh = plsc.VectorSubcoreMesh(
        core_axis_name="c", subcore_axis_name="s",
        num_cores=NUM_CORES, num_subcores=NUM_SUBCORES)

    scratch = (
        [pltpu.VMEM((unit,), jnp.int32) for _ in range(2)]
        + [pltpu.VMEM((unit, EMBED_DIM), jnp.float32) for _ in range(2)]
        + [pltpu.VMEM((4, cpw, SPAD, LPAD), jnp.float32) for _ in range(2)]
        + [pltpu.SemaphoreType.DMA for _ in range(6)]
    )

    @functools.partial(
        pl.kernel,
        out_type=jax.ShapeDtypeStruct((h, 4, nb // 128, 8, 128),
                                      jnp.float32),
        mesh=mesh,
        scratch_types=scratch,
        compiler_params=pltpu.CompilerParams(
            use_tc_tiling_on_sc=False, needs_layout_passes=False),
    )
    def gather_kernel(idx_hbm, table_hbm, out_hbm, *refs):
        idx_v = refs[0:2]
        rows_v = refs[2:4]
        t_v = refs[4:6]
        idx_sem = refs[6:8]
        gat_sem = refs[8:10]
        out_sem = refs[10:12]

        wid = lax.axis_index("s") * NUM_CORES + lax.axis_index("c")
        c0 = wid * cpw
        b0 = wid * unit

        iota = lax.iota(jnp.int32, 16)
        tv_lo = lax.shift_right_logical(iota, 3)
        tv_hi = tv_lo + 2
        sv = lax.bitwise_and(iota, 7)

        def idx_copy(i, b):
            pltpu.async_copy(
                idx_hbm.at[pl.ds(i * nb + b0, unit)], idx_v[b], idx_sem[b])

        def wait_idx(b):
            pltpu.make_async_copy(
                idx_hbm.at[pl.ds(0, unit)], idx_v[b], idx_sem[b]).wait()

        def gather_start(b):
            pltpu.async_copy(table_hbm.at[idx_v[b]], rows_v[b], gat_sem[b])

        def wait_gat(b):
            pltpu.make_async_copy(
                table_hbm.at[idx_v[b]], rows_v[b], gat_sem[b]).wait()

        def out_copy(i, b):
            for t in range(4):
                pltpu.async_copy(
                    t_v[b].at[t, pl.ds(0, cpw), pl.ds(0, 8), pl.ds(0, 128)],
                    out_hbm.at[i, t, pl.ds(c0, cpw)], out_sem[b])

        def wait_out(b):
            for t in range(4):
                pltpu.make_async_copy(
                    t_v[b].at[t, pl.ds(0, cpw), pl.ds(0, 8), pl.ds(0, 128)],
                    out_hbm.at[0, t, pl.ds(0, cpw)], out_sem[b]).wait()

        def transpose(b):
            def body(k, lv):
                cb = k // 8
                cbv = jnp.zeros((16,), jnp.int32) + cb
                r0 = k * 16
                for u in range(16):
                    v_lo = rows_v[b][r0 + u, pl.ds(0, 16)]
                    v_hi = rows_v[b][r0 + u, pl.ds(16, 16)]
                    plsc.store_scatter(t_v[b], [tv_lo, cbv, sv, lv], v_lo)
                    plsc.store_scatter(t_v[b], [tv_hi, cbv, sv, lv], v_hi)
                    lv = lv + 1
                return lax.bitwise_and(lv, 127)
            lax.fori_loop(0, cpw * 8, body, jnp.zeros((16,), jnp.int32))

        def step_a(i, b):
            wait_idx(b)
            gather_start(b)

        def step_b(j, pb, check_out, prefetch):
            wait_gat(pb)
            if check_out:
                wait_out(pb)
            transpose(pb)
            out_copy(j, pb)
            if prefetch:
                idx_copy(j + 2, pb)

        idx_copy(0, 0)
        idx_copy(1, 1)
        step_a(0, 0)
        step_a(1, 1)
        step_b(0, 0, False, True)
        step_a(2, 0)
        step_b(1, 1, False, True)

        def group(g, carry):
            i0 = 3 + 2 * g
            step_a(i0, 1)
            step_b(i0 - 1, 0, True, True)
            step_a(i0 + 1, 0)
            step_b(i0, 1, True, True)
            return carry

        lax.fori_loop(0, (n - 4) // 2, group, 0)

        step_a(n - 1, 1)
        step_b(n - 2, 0, True, False)
        step_b(n - 1, 1, True, False)
        wait_out(0)
        wait_out(1)

    return gather_kernel


def kernel(batch, word_embeddings):
    nb, h = batch.shape
    flat = batch.T.reshape(nb * h)
    L = _make_gather(nb, h)(flat, word_embeddings)
    return jnp.transpose(L, (2, 4, 0, 1, 3)).reshape(nb, h, EMBED_DIM)
